# baseline (device time: 269198 ns/iter reference)
import jax
import jax.numpy as jnp
from jax import lax
from jax.experimental import pallas as pl
from jax.experimental.pallas import tpu as pltpu

N_DEV = 8
N_LAYERS = 3


def kernel(x, Win0, Wout0, Win1, Wout1, Win2, Wout2):
    b, d = x.shape
    h = Win0.shape[1]

    def body(x_ref, win0_ref, wout0_ref, win1_ref, wout1_ref, win2_ref,
             wout2_ref, out_ref, winG, woutG, send_w, send_o, recv_w, recv_o):
        me = lax.axis_index("i")
        left = (me - 1) % N_DEV
        right = (me + 1) % N_DEV

        barrier_sem = pltpu.get_barrier_semaphore()
        for nbr in [left, right]:
            pl.semaphore_signal(
                barrier_sem, inc=1,
                device_id=(nbr,), device_id_type=pl.DeviceIdType.MESH,
            )
        pl.semaphore_wait(barrier_sem, 2)

        winG[0, 0, :, :] = win0_ref[:, :]
        winG[0, 1, :, :] = win1_ref[:, :]
        winG[0, 2, :, :] = win2_ref[:, :]
        woutG[0, 0, :, :] = wout0_ref[:, :]
        woutG[0, 1, :, :] = wout1_ref[:, :]
        woutG[0, 2, :, :] = wout2_ref[:, :]

        for hop in range(N_DEV - 1):
            rdma_w = pltpu.make_async_remote_copy(
                src_ref=winG.at[hop],
                dst_ref=winG.at[hop + 1],
                send_sem=send_w,
                recv_sem=recv_w.at[hop],
                device_id=(right,),
                device_id_type=pl.DeviceIdType.MESH,
            )
            rdma_o = pltpu.make_async_remote_copy(
                src_ref=woutG.at[hop],
                dst_ref=woutG.at[hop + 1],
                send_sem=send_o,
                recv_sem=recv_o.at[hop],
                device_id=(right,),
                device_id_type=pl.DeviceIdType.MESH,
            )
            rdma_w.start()
            rdma_o.start()
            rdma_w.wait()
            rdma_o.wait()

        x_cur = x_ref[:, :]
        for l in range(N_LAYERS):
            acc = jnp.zeros((b, d), dtype=jnp.float32)
            for k in range(N_DEV):
                hk = jnp.maximum(
                    jnp.dot(x_cur, winG[k, l, :, :],
                            preferred_element_type=jnp.float32),
                    0.0,
                )
                acc = acc + jnp.dot(hk, woutG[k, l, :, :],
                                    preferred_element_type=jnp.float32)
            x_cur = acc
        out_ref[:, :] = x_cur

    return pl.pallas_call(
        body,
        out_shape=jax.ShapeDtypeStruct((b, d), jnp.float32),
        in_specs=[pl.BlockSpec(memory_space=pltpu.VMEM)] * 7,
        out_specs=pl.BlockSpec(memory_space=pltpu.VMEM),
        scratch_shapes=[
            pltpu.VMEM((N_DEV, N_LAYERS, d, h), jnp.float32),
            pltpu.VMEM((N_DEV, N_LAYERS, h, d), jnp.float32),
            pltpu.SemaphoreType.DMA,
            pltpu.SemaphoreType.DMA,
            pltpu.SemaphoreType.DMA((N_DEV - 1,)),
            pltpu.SemaphoreType.DMA((N_DEV - 1,)),
        ],
        compiler_params=pltpu.CompilerParams(collective_id=0),
    )(x, Win0, Wout0, Win1, Wout1, Win2, Wout2)


# device time: 104609 ns/iter; 2.5734x vs baseline; 2.5734x over previous
import jax
import jax.numpy as jnp
from jax import lax
from jax.experimental import pallas as pl
from jax.experimental.pallas import tpu as pltpu

N_DEV = 8
N_LAYERS = 3
MASKS = (1, 3, 4)


def kernel(x, Win0, Wout0, Win1, Wout1, Win2, Wout2):
    b, d = x.shape
    h = Win0.shape[1]

    def body(x_ref, win0_ref, wout0_ref, win1_ref, wout1_ref, win2_ref,
             wout2_ref, out_ref, winG, woutG, send_w, send_o, recv_w, recv_o):
        me = lax.axis_index("i")

        barrier_sem = pltpu.get_barrier_semaphore()
        for m in MASKS:
            pl.semaphore_signal(
                barrier_sem, inc=1,
                device_id=(me ^ m,), device_id_type=pl.DeviceIdType.MESH,
            )
        pl.semaphore_wait(barrier_sem, len(MASKS))

        winG[0, 0, :, :] = win0_ref[:, :]
        winG[1, 0, :, :] = win1_ref[:, :]
        winG[2, 0, :, :] = win2_ref[:, :]
        woutG[0, 0, :, :] = wout0_ref[:, :]
        woutG[1, 0, :, :] = wout1_ref[:, :]
        woutG[2, 0, :, :] = wout2_ref[:, :]

        for r in range(3):
            n = 1 << r
            rdmas = []
            for l in range(N_LAYERS):
                mask = MASKS[(l + r) % 3]
                partner = me ^ mask
                for buf, ssem, rsem in (
                    (winG, send_w, recv_w),
                    (woutG, send_o, recv_o),
                ):
                    rdma = pltpu.make_async_remote_copy(
                        src_ref=buf.at[l, pl.ds(0, n)],
                        dst_ref=buf.at[l, pl.ds(n, n)],
                        send_sem=ssem.at[l],
                        recv_sem=rsem.at[r, l],
                        device_id=(partner,),
                        device_id_type=pl.DeviceIdType.MESH,
                    )
                    rdma.start()
                    rdmas.append(rdma)
            for rdma in rdmas:
                rdma.wait()

        x_cur = x_ref[:, :]
        for l in range(N_LAYERS):
            acc = jnp.zeros((b, d), dtype=jnp.float32)
            for t in range(N_DEV):
                hk = jnp.maximum(
                    jnp.dot(x_cur, winG[l, t, :, :],
                            preferred_element_type=jnp.float32),
                    0.0,
                )
                acc = acc + jnp.dot(hk, woutG[l, t, :, :],
                                    preferred_element_type=jnp.float32)
            x_cur = acc
        out_ref[:, :] = x_cur

    return pl.pallas_call(
        body,
        out_shape=jax.ShapeDtypeStruct((b, d), jnp.float32),
        in_specs=[pl.BlockSpec(memory_space=pltpu.VMEM)] * 7,
        out_specs=pl.BlockSpec(memory_space=pltpu.VMEM),
        scratch_shapes=[
            pltpu.VMEM((N_LAYERS, N_DEV, d, h), jnp.float32),
            pltpu.VMEM((N_LAYERS, N_DEV, h, d), jnp.float32),
            pltpu.SemaphoreType.DMA((N_LAYERS,)),
            pltpu.SemaphoreType.DMA((N_LAYERS,)),
            pltpu.SemaphoreType.DMA((3, N_LAYERS)),
            pltpu.SemaphoreType.DMA((3, N_LAYERS)),
        ],
        compiler_params=pltpu.CompilerParams(collective_id=0),
    )(x, Win0, Wout0, Win1, Wout1, Win2, Wout2)


# device time: 65296 ns/iter; 4.1227x vs baseline; 1.6021x over previous
import jax
import jax.numpy as jnp
from jax import lax
from jax.experimental import pallas as pl
from jax.experimental.pallas import tpu as pltpu

N_DEV = 8
N_LAYERS = 3
MASKS = (1, 3, 4)


def kernel(x, Win0, Wout0, Win1, Wout1, Win2, Wout2):
    b, d = x.shape
    h = Win0.shape[1]

    def body(x_ref, win0_ref, wout0_ref, win1_ref, wout1_ref, win2_ref,
             wout2_ref, out_ref, winG, woutG, send_w, send_o, recv_w, recv_o):
        me = lax.axis_index("i")

        barrier_sem = pltpu.get_barrier_semaphore()
        for m in MASKS:
            pl.semaphore_signal(
                barrier_sem, inc=1,
                device_id=(me ^ m,), device_id_type=pl.DeviceIdType.MESH,
            )
        pl.semaphore_wait(barrier_sem, len(MASKS))

        winG[0, 0, :, :] = win0_ref[:, :].astype(jnp.bfloat16)
        winG[1, 0, :, :] = win1_ref[:, :].astype(jnp.bfloat16)
        winG[2, 0, :, :] = win2_ref[:, :].astype(jnp.bfloat16)
        woutG[0, 0, :, :] = wout0_ref[:, :].astype(jnp.bfloat16)
        woutG[1, 0, :, :] = wout1_ref[:, :].astype(jnp.bfloat16)
        woutG[2, 0, :, :] = wout2_ref[:, :].astype(jnp.bfloat16)

        for r in range(3):
            n = 1 << r
            rdmas = []
            for l in range(N_LAYERS):
                mask = MASKS[(l + r) % 3]
                partner = me ^ mask
                for buf, ssem, rsem in (
                    (winG, send_w, recv_w),
                    (woutG, send_o, recv_o),
                ):
                    rdma = pltpu.make_async_remote_copy(
                        src_ref=buf.at[l, pl.ds(0, n)],
                        dst_ref=buf.at[l, pl.ds(n, n)],
                        send_sem=ssem.at[l],
                        recv_sem=rsem.at[r, l],
                        device_id=(partner,),
                        device_id_type=pl.DeviceIdType.MESH,
                    )
                    rdma.start()
                    rdmas.append(rdma)
            for rdma in rdmas:
                rdma.wait()

        x_cur = x_ref[:, :]
        for l in range(N_LAYERS):
            x_bf = x_cur.astype(jnp.bfloat16)
            acc = jnp.zeros((b, d), dtype=jnp.float32)
            for t in range(N_DEV):
                hk = jnp.maximum(
                    jnp.dot(x_bf, winG[l, t, :, :],
                            preferred_element_type=jnp.float32),
                    0.0,
                ).astype(jnp.bfloat16)
                acc = acc + jnp.dot(hk, woutG[l, t, :, :],
                                    preferred_element_type=jnp.float32)
            x_cur = acc
        out_ref[:, :] = x_cur

    return pl.pallas_call(
        body,
        out_shape=jax.ShapeDtypeStruct((b, d), jnp.float32),
        in_specs=[pl.BlockSpec(memory_space=pltpu.VMEM)] * 7,
        out_specs=pl.BlockSpec(memory_space=pltpu.VMEM),
        scratch_shapes=[
            pltpu.VMEM((N_LAYERS, N_DEV, d, h), jnp.bfloat16),
            pltpu.VMEM((N_LAYERS, N_DEV, h, d), jnp.bfloat16),
            pltpu.SemaphoreType.DMA((N_LAYERS,)),
            pltpu.SemaphoreType.DMA((N_LAYERS,)),
            pltpu.SemaphoreType.DMA((3, N_LAYERS)),
            pltpu.SemaphoreType.DMA((3, N_LAYERS)),
        ],
        compiler_params=pltpu.CompilerParams(collective_id=0),
    )(x, Win0, Wout0, Win1, Wout1, Win2, Wout2)


# device time: 62483 ns/iter; 4.3083x vs baseline; 1.0450x over previous
import jax
import jax.numpy as jnp
from jax import lax
from jax.experimental import pallas as pl
from jax.experimental.pallas import tpu as pltpu

N_DEV = 8
N_LAYERS = 3
MASKS = (1, 3, 4)


def kernel(x, Win0, Wout0, Win1, Wout1, Win2, Wout2):
    b, d = x.shape
    h = Win0.shape[1]

    def body(x_ref, win0_ref, wout0_ref, win1_ref, wout1_ref, win2_ref,
             wout2_ref, out_ref, winG, woutG, send_w, send_o, recv_w, recv_o):
        me = lax.axis_index("i")

        barrier_sem = pltpu.get_barrier_semaphore()
        for m in MASKS:
            pl.semaphore_signal(
                barrier_sem, inc=1,
                device_id=(me ^ m,), device_id_type=pl.DeviceIdType.MESH,
            )
        pl.semaphore_wait(barrier_sem, len(MASKS))

        winG[0, 0, :, :] = win0_ref[:, :].astype(jnp.bfloat16)
        winG[1, 0, :, :] = win1_ref[:, :].astype(jnp.bfloat16)
        winG[2, 0, :, :] = win2_ref[:, :].astype(jnp.bfloat16)
        woutG[0, 0, :, :] = wout0_ref[:, :].astype(jnp.bfloat16)
        woutG[1, 0, :, :] = wout1_ref[:, :].astype(jnp.bfloat16)
        woutG[2, 0, :, :] = wout2_ref[:, :].astype(jnp.bfloat16)

        tensors = ((winG, send_w, recv_w), (woutG, send_o, recv_o))

        def copy(buf, l, src0, dst0, n, ssem, rsem_idx, partner):
            rsem = (recv_w if ssem is send_w else recv_o)
            return pltpu.make_async_remote_copy(
                src_ref=buf.at[l, pl.ds(src0, n)],
                dst_ref=buf.at[l, pl.ds(dst0, n)],
                send_sem=ssem.at[l],
                recv_sem=rsem.at[rsem_idx, l],
                device_id=(partner,),
                device_id_type=pl.DeviceIdType.MESH,
            )

        partner = [[me ^ MASKS[(l + r) % 3] for r in range(3)]
                   for l in range(N_LAYERS)]

        sends = []
        grp = {}

        def issue(l, r, ti, src0, dst0, n):
            buf, ssem, _ = tensors[ti]
            c = copy(buf, l, src0, dst0, n, ssem, r, partner[l][r])
            c.start()
            sends.append(c)
            grp.setdefault((r, l, ti), []).append(c)

        for r, dst0 in ((0, 1), (1, 2), (2, 4)):
            for l in range(N_LAYERS):
                for ti in range(2):
                    issue(l, r, ti, 0, dst0, 1)

        x_bf = x_ref[:, :].astype(jnp.bfloat16)
        acc0 = jnp.zeros((b, d), dtype=jnp.float32)

        def term(l, t, xb):
            hk = jnp.maximum(
                jnp.dot(xb, winG[l, t, :, :],
                        preferred_element_type=jnp.float32),
                0.0,
            ).astype(jnp.bfloat16)
            return jnp.dot(hk, woutG[l, t, :, :],
                           preferred_element_type=jnp.float32)

        for l in range(N_LAYERS):
            for ti in range(2):
                for c in grp[(0, l, ti)]:
                    c.wait_recv()
            for ti in range(2):
                issue(l, 1, ti, 1, 3, 1)
                issue(l, 2, ti, 1, 5, 1)
        acc0 = acc0 + term(0, 0, x_bf) + term(0, 1, x_bf)

        for l in range(N_LAYERS):
            for ti in range(2):
                for c in grp[(1, l, ti)]:
                    c.wait_recv()
            for ti in range(2):
                issue(l, 2, ti, 2, 6, 2)
        acc0 = acc0 + term(0, 2, x_bf) + term(0, 3, x_bf)

        for l in range(N_LAYERS):
            for ti in range(2):
                for c in grp[(2, l, ti)]:
                    c.wait_recv()

        for t in range(4, N_DEV):
            acc0 = acc0 + term(0, t, x_bf)
        x_cur = acc0
        for l in range(1, N_LAYERS):
            xb = x_cur.astype(jnp.bfloat16)
            acc = jnp.zeros((b, d), dtype=jnp.float32)
            for t in range(N_DEV):
                acc = acc + term(l, t, xb)
            x_cur = acc
        out_ref[:, :] = x_cur

        for c in sends:
            c.wait_send()

    return pl.pallas_call(
        body,
        out_shape=jax.ShapeDtypeStruct((b, d), jnp.float32),
        in_specs=[pl.BlockSpec(memory_space=pltpu.VMEM)] * 7,
        out_specs=pl.BlockSpec(memory_space=pltpu.VMEM),
        scratch_shapes=[
            pltpu.VMEM((N_LAYERS, N_DEV, d, h), jnp.bfloat16),
            pltpu.VMEM((N_LAYERS, N_DEV, h, d), jnp.bfloat16),
            pltpu.SemaphoreType.DMA((N_LAYERS,)),
            pltpu.SemaphoreType.DMA((N_LAYERS,)),
            pltpu.SemaphoreType.DMA((3, N_LAYERS)),
            pltpu.SemaphoreType.DMA((3, N_LAYERS)),
        ],
        compiler_params=pltpu.CompilerParams(collective_id=0),
    )(x, Win0, Wout0, Win1, Wout1, Win2, Wout2)


# device time: 45991 ns/iter; 5.8533x vs baseline; 1.3586x over previous
import jax
import jax.numpy as jnp
from jax import lax
from jax.experimental import pallas as pl
from jax.experimental.pallas import tpu as pltpu

N_DEV = 8
N_LAYERS = 3
MASKS = (1, 3, 4)


_BLK = 32


def _quant(w):
    r, c = w.shape
    wr = w.reshape(r // _BLK, _BLK, c)
    m = jnp.max(jnp.abs(wr), axis=1, keepdims=True)
    s = jnp.maximum(m, 1e-20) / 127.0
    q = jnp.clip(jnp.round(wr / s), -127, 127).astype(jnp.int8)
    return q.reshape(r, c), s.squeeze(1).astype(jnp.float32)


def kernel(x, Win0, Wout0, Win1, Wout1, Win2, Wout2):
    b, d = x.shape
    h = Win0.shape[1]

    winQ, winS, woutQ, woutS = [], [], [], []
    for wi, wo in ((Win0, Wout0), (Win1, Wout1), (Win2, Wout2)):
        q, s = _quant(wi)
        winQ.append(q)
        winS.append(s)
        q, s = _quant(wo)
        woutQ.append(q)
        woutS.append(s)

    def body(x_ref, wq0, ws0, oq0, os0, wq1, ws1, oq1, os1, wq2, ws2, oq2,
             os2, out_ref, winG, woutG, swinG, swoutG,
             send_w, send_o, recv_w, recv_o):
        me = lax.axis_index("i")

        barrier_sem = pltpu.get_barrier_semaphore()
        for m in MASKS:
            pl.semaphore_signal(
                barrier_sem, inc=1,
                device_id=(me ^ m,), device_id_type=pl.DeviceIdType.MESH,
            )
        pl.semaphore_wait(barrier_sem, len(MASKS))

        for l, (wq, ws, oq, osc) in enumerate(
            ((wq0, ws0, oq0, os0), (wq1, ws1, oq1, os1), (wq2, ws2, oq2, os2))
        ):
            winG[l, 0, :, :] = wq[:, :]
            swinG[l, 0, :, :] = ws[:, :]
            woutG[l, 0, :, :] = oq[:, :]
            swoutG[l, 0, :, :] = osc[:, :]

        tensors = ((winG, swinG, send_w, recv_w),
                   (woutG, swoutG, send_o, recv_o))

        partner = [[me ^ MASKS[(l + r) % 3] for r in range(3)]
                   for l in range(N_LAYERS)]

        sends = []
        grp = {}

        def issue(l, r, ti, src0, dst0, n):
            qbuf, sbuf, ssem, rsem = tensors[ti]
            for buf in (qbuf, sbuf):
                c = pltpu.make_async_remote_copy(
                    src_ref=buf.at[l, pl.ds(src0, n)],
                    dst_ref=buf.at[l, pl.ds(dst0, n)],
                    send_sem=ssem.at[l],
                    recv_sem=rsem.at[r, l],
                    device_id=(partner[l][r],),
                    device_id_type=pl.DeviceIdType.MESH,
                )
                c.start()
                sends.append(c)
                grp.setdefault((r, l, ti), []).append(c)

        for r, dst0 in ((0, 1), (1, 2), (2, 4)):
            for l in range(N_LAYERS):
                for ti in range(2):
                    issue(l, r, ti, 0, dst0, 1)

        x_bf = x_ref[:, :].astype(jnp.bfloat16)
        acc0 = jnp.zeros((b, d), dtype=jnp.float32)

        def dequant(qbuf, sbuf, l, t, rows, cols):
            q = qbuf[l, t, :, :].astype(jnp.bfloat16)
            s = sbuf[l, t, :, :].astype(jnp.bfloat16)
            w = q.reshape(rows // _BLK, _BLK, cols) * s[:, None, :]
            return w.reshape(rows, cols)

        def term(l, t, xb):
            wbf = dequant(winG, swinG, l, t, d, h)
            hk = jnp.maximum(
                jnp.dot(xb, wbf, preferred_element_type=jnp.float32),
                0.0,
            ).astype(jnp.bfloat16)
            obf = dequant(woutG, swoutG, l, t, h, d)
            return jnp.dot(hk, obf, preferred_element_type=jnp.float32)

        for l in range(N_LAYERS):
            for ti in range(2):
                for c in grp[(0, l, ti)]:
                    c.wait_recv()
            for ti in range(2):
                issue(l, 1, ti, 1, 3, 1)
                issue(l, 2, ti, 1, 5, 1)
        acc0 = acc0 + term(0, 0, x_bf) + term(0, 1, x_bf)

        for l in range(N_LAYERS):
            for ti in range(2):
                for c in grp[(1, l, ti)]:
                    c.wait_recv()
            for ti in range(2):
                issue(l, 2, ti, 2, 6, 2)
        acc0 = acc0 + term(0, 2, x_bf) + term(0, 3, x_bf)

        for ti in range(2):
            for c in grp[(2, 0, ti)]:
                c.wait_recv()
        for t in range(4, N_DEV):
            acc0 = acc0 + term(0, t, x_bf)
        x_cur = acc0
        for l in range(1, N_LAYERS):
            for ti in range(2):
                for c in grp[(2, l, ti)]:
                    c.wait_recv()
            xb = x_cur.astype(jnp.bfloat16)
            acc = jnp.zeros((b, d), dtype=jnp.float32)
            for t in range(N_DEV):
                acc = acc + term(l, t, xb)
            x_cur = acc
        out_ref[:, :] = x_cur

        for c in sends:
            c.wait_send()

    return pl.pallas_call(
        body,
        out_shape=jax.ShapeDtypeStruct((b, d), jnp.float32),
        in_specs=[pl.BlockSpec(memory_space=pltpu.VMEM)] * 13,
        out_specs=pl.BlockSpec(memory_space=pltpu.VMEM),
        scratch_shapes=[
            pltpu.VMEM((N_LAYERS, N_DEV, d, h), jnp.int8),
            pltpu.VMEM((N_LAYERS, N_DEV, h, d), jnp.int8),
            pltpu.VMEM((N_LAYERS, N_DEV, d // _BLK, h), jnp.float32),
            pltpu.VMEM((N_LAYERS, N_DEV, h // _BLK, d), jnp.float32),
            pltpu.SemaphoreType.DMA((N_LAYERS,)),
            pltpu.SemaphoreType.DMA((N_LAYERS,)),
            pltpu.SemaphoreType.DMA((3, N_LAYERS)),
            pltpu.SemaphoreType.DMA((3, N_LAYERS)),
        ],
        compiler_params=pltpu.CompilerParams(collective_id=0),
    )(x, winQ[0], winS[0], woutQ[0], woutS[0],
      winQ[1], winS[1], woutQ[1], woutS[1],
      winQ[2], winS[2], woutQ[2], woutS[2])


# device time: 45139 ns/iter; 5.9638x vs baseline; 1.0189x over previous
import jax
import jax.numpy as jnp
from jax import lax
from jax.experimental import pallas as pl
from jax.experimental.pallas import tpu as pltpu

N_DEV = 8
N_LAYERS = 3
MASKS = (1, 3, 4)


_BLK = 32


def _quant(w):
    r, c = w.shape
    wr = w.reshape(r // _BLK, _BLK, c)
    m = jnp.max(jnp.abs(wr), axis=1, keepdims=True)
    s = jnp.maximum(m, 1e-20) / 127.0
    q = jnp.clip(jnp.round(wr / s), -127, 127).astype(jnp.int8)
    return q.reshape(r, c), s.squeeze(1).astype(jnp.float32)


def kernel(x, Win0, Wout0, Win1, Wout1, Win2, Wout2):
    b, d = x.shape
    h = Win0.shape[1]

    winQ, winS, woutQ, woutS = [], [], [], []
    for wi, wo in ((Win0, Wout0), (Win1, Wout1), (Win2, Wout2)):
        q, s = _quant(wi)
        winQ.append(q)
        winS.append(s)
        q, s = _quant(wo)
        woutQ.append(q)
        woutS.append(s)

    def body(x_ref, wq0, ws0, oq0, os0, wq1, ws1, oq1, os1, wq2, ws2, oq2,
             os2, out_ref, winG, woutG, swinG, swoutG, winB, woutB,
             send_w, send_o, recv_w, recv_o):
        me = lax.axis_index("i")

        barrier_sem = pltpu.get_barrier_semaphore()
        for m in MASKS:
            pl.semaphore_signal(
                barrier_sem, inc=1,
                device_id=(me ^ m,), device_id_type=pl.DeviceIdType.MESH,
            )
        pl.semaphore_wait(barrier_sem, len(MASKS))

        for l, (wq, ws, oq, osc) in enumerate(
            ((wq0, ws0, oq0, os0), (wq1, ws1, oq1, os1), (wq2, ws2, oq2, os2))
        ):
            winG[l, 0, :, :] = wq[:, :]
            swinG[l, 0, :, :] = ws[:, :]
            woutG[l, 0, :, :] = oq[:, :]
            swoutG[l, 0, :, :] = osc[:, :]

        tensors = ((winG, swinG, send_w, recv_w),
                   (woutG, swoutG, send_o, recv_o))

        partner = [[me ^ MASKS[(l + r) % 3] for r in range(3)]
                   for l in range(N_LAYERS)]

        sends = []
        grp = {}

        def issue(l, r, ti, src0, dst0, n):
            qbuf, sbuf, ssem, rsem = tensors[ti]
            for buf in (qbuf, sbuf):
                c = pltpu.make_async_remote_copy(
                    src_ref=buf.at[l, pl.ds(src0, n)],
                    dst_ref=buf.at[l, pl.ds(dst0, n)],
                    send_sem=ssem.at[l],
                    recv_sem=rsem.at[r, l],
                    device_id=(partner[l][r],),
                    device_id_type=pl.DeviceIdType.MESH,
                )
                c.start()
                sends.append(c)
                grp.setdefault((r, l, ti), []).append(c)

        for r, dst0 in ((0, 1), (1, 2), (2, 4)):
            for l in range(N_LAYERS):
                for ti in range(2):
                    issue(l, r, ti, 0, dst0, 1)

        x_bf = x_ref[:, :].astype(jnp.bfloat16)
        acc0 = jnp.zeros((b, d), dtype=jnp.float32)

        def dequant(qbuf, sbuf, l, t, rows, cols):
            q = qbuf[l, t, :, :].astype(jnp.bfloat16)
            s = sbuf[l, t, :, :].astype(jnp.bfloat16)
            w = q.reshape(rows // _BLK, _BLK, cols) * s[:, None, :]
            return w.reshape(rows, cols)

        def convert(l, t):
            winB[l, t, :, :] = dequant(winG, swinG, l, t, d, h)
            woutB[l, t, :, :] = dequant(woutG, swoutG, l, t, h, d)

        def term(l, t, xb):
            hk = jnp.maximum(
                jnp.dot(xb, winB[l, t, :, :],
                        preferred_element_type=jnp.float32),
                0.0,
            ).astype(jnp.bfloat16)
            return jnp.dot(hk, woutB[l, t, :, :],
                           preferred_element_type=jnp.float32)

        for l in range(N_LAYERS):
            convert(l, 0)

        for l in range(N_LAYERS):
            for ti in range(2):
                for c in grp[(0, l, ti)]:
                    c.wait_recv()
            for ti in range(2):
                issue(l, 1, ti, 1, 3, 1)
                issue(l, 2, ti, 1, 5, 1)
            convert(l, 1)
        acc0 = acc0 + term(0, 0, x_bf) + term(0, 1, x_bf)

        for l in range(N_LAYERS):
            for ti in range(2):
                for c in grp[(1, l, ti)]:
                    c.wait_recv()
            for ti in range(2):
                issue(l, 2, ti, 2, 6, 2)
            convert(l, 2)
            convert(l, 3)
        acc0 = acc0 + term(0, 2, x_bf) + term(0, 3, x_bf)

        for ti in range(2):
            for c in grp[(2, 0, ti)]:
                c.wait_recv()
        for t in range(4, N_DEV):
            convert(0, t)
        for t in range(4, N_DEV):
            acc0 = acc0 + term(0, t, x_bf)
        x_cur = acc0
        for l in range(1, N_LAYERS):
            xb = x_cur.astype(jnp.bfloat16)
            acc = jnp.zeros((b, d), dtype=jnp.float32)
            for t in range(4):
                acc = acc + term(l, t, xb)
            for ti in range(2):
                for c in grp[(2, l, ti)]:
                    c.wait_recv()
            for t in range(4, N_DEV):
                convert(l, t)
            for t in range(4, N_DEV):
                acc = acc + term(l, t, xb)
            x_cur = acc
        out_ref[:, :] = x_cur

        for c in sends:
            c.wait_send()

    return pl.pallas_call(
        body,
        out_shape=jax.ShapeDtypeStruct((b, d), jnp.float32),
        in_specs=[pl.BlockSpec(memory_space=pltpu.VMEM)] * 13,
        out_specs=pl.BlockSpec(memory_space=pltpu.VMEM),
        scratch_shapes=[
            pltpu.VMEM((N_LAYERS, N_DEV, d, h), jnp.int8),
            pltpu.VMEM((N_LAYERS, N_DEV, h, d), jnp.int8),
            pltpu.VMEM((N_LAYERS, N_DEV, d // _BLK, h), jnp.float32),
            pltpu.VMEM((N_LAYERS, N_DEV, h // _BLK, d), jnp.float32),
            pltpu.VMEM((N_LAYERS, N_DEV, d, h), jnp.bfloat16),
            pltpu.VMEM((N_LAYERS, N_DEV, h, d), jnp.bfloat16),
            pltpu.SemaphoreType.DMA((N_LAYERS,)),
            pltpu.SemaphoreType.DMA((N_LAYERS,)),
            pltpu.SemaphoreType.DMA((3, N_LAYERS)),
            pltpu.SemaphoreType.DMA((3, N_LAYERS)),
        ],
        compiler_params=pltpu.CompilerParams(collective_id=0),
    )(x, winQ[0], winS[0], woutQ[0], woutS[0],
      winQ[1], winS[1], woutQ[1], woutS[1],
      winQ[2], winS[2], woutQ[2], woutS[2])


# device time: 43883 ns/iter; 6.1344x vs baseline; 1.0286x over previous
import jax
import jax.numpy as jnp
from jax import lax
from jax.experimental import pallas as pl
from jax.experimental.pallas import tpu as pltpu

N_DEV = 8
N_LAYERS = 3
MASKS = (1, 3, 4)


_BLK = 32


def _quant(w):
    r, c = w.shape
    wr = w.reshape(r // _BLK, _BLK, c)
    m = jnp.max(jnp.abs(wr), axis=1, keepdims=True)
    s = (jnp.maximum(m, 1e-20) / 126.0).astype(jnp.bfloat16)
    q = jnp.clip(jnp.round(wr / s.astype(jnp.float32)), -127, 127)
    return q.astype(jnp.int8).reshape(r, c), s.squeeze(1)


def kernel(x, Win0, Wout0, Win1, Wout1, Win2, Wout2):
    b, d = x.shape
    h = Win0.shape[1]

    winQ, winS, woutQ, woutS = [], [], [], []
    for wi, wo in ((Win0, Wout0), (Win1, Wout1), (Win2, Wout2)):
        q, s = _quant(wi)
        winQ.append(q)
        winS.append(s)
        q, s = _quant(wo)
        woutQ.append(q)
        woutS.append(s)

    def body(x_ref, wq0, ws0, oq0, os0, wq1, ws1, oq1, os1, wq2, ws2, oq2,
             os2, out_ref, winG, woutG, swinG, swoutG, winB, woutB,
             send_w, send_o, recv_w, recv_o):
        me = lax.axis_index("i")

        barrier_sem = pltpu.get_barrier_semaphore()
        for m in MASKS:
            pl.semaphore_signal(
                barrier_sem, inc=1,
                device_id=(me ^ m,), device_id_type=pl.DeviceIdType.MESH,
            )
        pl.semaphore_wait(barrier_sem, len(MASKS))

        for l, (wq, ws, oq, osc) in enumerate(
            ((wq0, ws0, oq0, os0), (wq1, ws1, oq1, os1), (wq2, ws2, oq2, os2))
        ):
            winG[l, 0, :, :] = wq[:, :]
            swinG[l, 0, :, :] = ws[:, :]
            woutG[l, 0, :, :] = oq[:, :]
            swoutG[l, 0, :, :] = osc[:, :]

        tensors = ((winG, swinG, send_w, recv_w),
                   (woutG, swoutG, send_o, recv_o))

        partner = [[me ^ MASKS[(l + r) % 3] for r in range(3)]
                   for l in range(N_LAYERS)]

        sends = []
        grp = {}

        def issue(l, r, ti, src0, dst0, n):
            qbuf, sbuf, ssem, rsem = tensors[ti]
            for buf in (qbuf, sbuf):
                c = pltpu.make_async_remote_copy(
                    src_ref=buf.at[l, pl.ds(src0, n)],
                    dst_ref=buf.at[l, pl.ds(dst0, n)],
                    send_sem=ssem.at[l],
                    recv_sem=rsem.at[r, l],
                    device_id=(partner[l][r],),
                    device_id_type=pl.DeviceIdType.MESH,
                )
                c.start()
                sends.append(c)
                grp.setdefault((r, l, ti), []).append(c)

        for r, dst0 in ((0, 1), (1, 2), (2, 4)):
            for l in range(N_LAYERS):
                for ti in range(2):
                    issue(l, r, ti, 0, dst0, 1)

        x_bf = x_ref[:, :].astype(jnp.bfloat16)
        acc0 = jnp.zeros((b, d), dtype=jnp.float32)

        def dequant(qbuf, sbuf, l, t, rows, cols):
            q = qbuf[l, t, :, :].astype(jnp.bfloat16)
            s = sbuf[l, t, :, :].astype(jnp.bfloat16)
            w = q.reshape(rows // _BLK, _BLK, cols) * s[:, None, :]
            return w.reshape(rows, cols)

        def convert(l, t):
            winB[l, t, :, :] = dequant(winG, swinG, l, t, d, h)
            woutB[l, t, :, :] = dequant(woutG, swoutG, l, t, h, d)

        def term(l, t, xb):
            hk = jnp.maximum(
                jnp.dot(xb, winB[l, t, :, :],
                        preferred_element_type=jnp.float32),
                0.0,
            ).astype(jnp.bfloat16)
            return jnp.dot(hk, woutB[l, t, :, :],
                           preferred_element_type=jnp.float32)

        for l in range(N_LAYERS):
            convert(l, 0)

        for l in range(N_LAYERS):
            for ti in range(2):
                for c in grp[(0, l, ti)]:
                    c.wait_recv()
            for ti in range(2):
                issue(l, 1, ti, 1, 3, 1)
                issue(l, 2, ti, 1, 5, 1)
            convert(l, 1)
        acc0 = acc0 + term(0, 0, x_bf) + term(0, 1, x_bf)

        for l in range(N_LAYERS):
            for ti in range(2):
                for c in grp[(1, l, ti)]:
                    c.wait_recv()
            for ti in range(2):
                issue(l, 2, ti, 2, 6, 2)
            convert(l, 2)
            convert(l, 3)
        acc0 = acc0 + term(0, 2, x_bf) + term(0, 3, x_bf)

        for ti in range(2):
            for c in grp[(2, 0, ti)]:
                c.wait_recv()
        for t in range(4, N_DEV):
            convert(0, t)
        for t in range(4, N_DEV):
            acc0 = acc0 + term(0, t, x_bf)
        x_cur = acc0
        for l in range(1, N_LAYERS):
            xb = x_cur.astype(jnp.bfloat16)
            acc = jnp.zeros((b, d), dtype=jnp.float32)
            for t in range(4):
                acc = acc + term(l, t, xb)
            for ti in range(2):
                for c in grp[(2, l, ti)]:
                    c.wait_recv()
            for t in range(4, N_DEV):
                convert(l, t)
            for t in range(4, N_DEV):
                acc = acc + term(l, t, xb)
            x_cur = acc
        out_ref[:, :] = x_cur

        for c in sends:
            c.wait_send()

    return pl.pallas_call(
        body,
        out_shape=jax.ShapeDtypeStruct((b, d), jnp.float32),
        in_specs=[pl.BlockSpec(memory_space=pltpu.VMEM)] * 13,
        out_specs=pl.BlockSpec(memory_space=pltpu.VMEM),
        scratch_shapes=[
            pltpu.VMEM((N_LAYERS, N_DEV, d, h), jnp.int8),
            pltpu.VMEM((N_LAYERS, N_DEV, h, d), jnp.int8),
            pltpu.VMEM((N_LAYERS, N_DEV, d // _BLK, h), jnp.bfloat16),
            pltpu.VMEM((N_LAYERS, N_DEV, h // _BLK, d), jnp.bfloat16),
            pltpu.VMEM((N_LAYERS, N_DEV, d, h), jnp.bfloat16),
            pltpu.VMEM((N_LAYERS, N_DEV, h, d), jnp.bfloat16),
            pltpu.SemaphoreType.DMA((N_LAYERS,)),
            pltpu.SemaphoreType.DMA((N_LAYERS,)),
            pltpu.SemaphoreType.DMA((3, N_LAYERS)),
            pltpu.SemaphoreType.DMA((3, N_LAYERS)),
        ],
        compiler_params=pltpu.CompilerParams(collective_id=0),
    )(x, winQ[0], winS[0], woutQ[0], woutS[0],
      winQ[1], winS[1], woutQ[1], woutS[1],
      winQ[2], winS[2], woutQ[2], woutS[2])


# device time: 42474 ns/iter; 6.3379x vs baseline; 1.0332x over previous
import jax
import jax.numpy as jnp
from jax import lax
from jax.experimental import pallas as pl
from jax.experimental.pallas import tpu as pltpu

N_DEV = 8
N_LAYERS = 3
MASKS = (1, 3, 4)


_BLK = 32


def _quant(w):
    r, c = w.shape
    wr = w.reshape(r // _BLK, _BLK, c)
    m = jnp.max(jnp.abs(wr), axis=1, keepdims=True)
    s = (jnp.maximum(m, 1e-20) / 126.0).astype(jnp.bfloat16)
    q = jnp.clip(jnp.round(wr / s.astype(jnp.float32)), -127, 127)
    return q.astype(jnp.int8).reshape(r, c), s.squeeze(1)


def kernel(x, Win0, Wout0, Win1, Wout1, Win2, Wout2):
    b, d = x.shape
    h = Win0.shape[1]

    def body(x_ref, win0_ref, wout0_ref, win1_ref, wout1_ref, win2_ref,
             wout2_ref, out_ref, winG, woutG, swinG, swoutG, winB, woutB,
             send_w, send_o, recv_w, recv_o):
        me = lax.axis_index("i")

        barrier_sem = pltpu.get_barrier_semaphore()
        for m in MASKS:
            pl.semaphore_signal(
                barrier_sem, inc=1,
                device_id=(me ^ m,), device_id_type=pl.DeviceIdType.MESH,
            )
        pl.semaphore_wait(barrier_sem, len(MASKS))

        tensors = ((winG, swinG, send_w, recv_w),
                   (woutG, swoutG, send_o, recv_o))

        partner = [[me ^ MASKS[(l + r) % 3] for r in range(3)]
                   for l in range(N_LAYERS)]

        sends = []
        grp = {}

        def issue(l, r, ti, src0, dst0, n):
            qbuf, sbuf, ssem, rsem = tensors[ti]
            for buf in (qbuf, sbuf):
                c = pltpu.make_async_remote_copy(
                    src_ref=buf.at[l, pl.ds(src0, n)],
                    dst_ref=buf.at[l, pl.ds(dst0, n)],
                    send_sem=ssem.at[l],
                    recv_sem=rsem.at[r, l],
                    device_id=(partner[l][r],),
                    device_id_type=pl.DeviceIdType.MESH,
                )
                c.start()
                sends.append(c)
                grp.setdefault((r, l, ti), []).append(c)

        for l, (wi_ref, wo_ref) in enumerate(
            ((win0_ref, wout0_ref), (win1_ref, wout1_ref),
             (win2_ref, wout2_ref))
        ):
            q, s = _quant(wi_ref[:, :])
            winG[l, 0, :, :] = q
            swinG[l, 0, :, :] = s
            q, s = _quant(wo_ref[:, :])
            woutG[l, 0, :, :] = q
            swoutG[l, 0, :, :] = s
            for r, dst0 in ((0, 1), (1, 2), (2, 4)):
                for ti in range(2):
                    issue(l, r, ti, 0, dst0, 1)

        x_bf = x_ref[:, :].astype(jnp.bfloat16)
        acc0 = jnp.zeros((b, d), dtype=jnp.float32)

        def dequant(qbuf, sbuf, l, t, rows, cols):
            q = qbuf[l, t, :, :].astype(jnp.bfloat16)
            s = sbuf[l, t, :, :].astype(jnp.bfloat16)
            w = q.reshape(rows // _BLK, _BLK, cols) * s[:, None, :]
            return w.reshape(rows, cols)

        def convert(l, t):
            winB[l, t, :, :] = dequant(winG, swinG, l, t, d, h)
            woutB[l, t, :, :] = dequant(woutG, swoutG, l, t, h, d)

        def term(l, t, xb):
            hk = jnp.maximum(
                jnp.dot(xb, winB[l, t, :, :],
                        preferred_element_type=jnp.float32),
                0.0,
            ).astype(jnp.bfloat16)
            return jnp.dot(hk, woutB[l, t, :, :],
                           preferred_element_type=jnp.float32)

        for l in range(N_LAYERS):
            convert(l, 0)

        for l in range(N_LAYERS):
            for ti in range(2):
                for c in grp[(0, l, ti)]:
                    c.wait_recv()
            for ti in range(2):
                issue(l, 1, ti, 1, 3, 1)
                issue(l, 2, ti, 1, 5, 1)
            convert(l, 1)
        acc0 = acc0 + term(0, 0, x_bf) + term(0, 1, x_bf)

        for l in range(N_LAYERS):
            for ti in range(2):
                for c in grp[(1, l, ti)]:
                    c.wait_recv()
            for ti in range(2):
                issue(l, 2, ti, 2, 6, 2)
            convert(l, 2)
            convert(l, 3)
        acc0 = acc0 + term(0, 2, x_bf) + term(0, 3, x_bf)

        for ti in range(2):
            for c in grp[(2, 0, ti)]:
                c.wait_recv()
        for t in range(4, N_DEV):
            convert(0, t)
        for t in range(4, N_DEV):
            acc0 = acc0 + term(0, t, x_bf)
        x_cur = acc0
        for l in range(1, N_LAYERS):
            xb = x_cur.astype(jnp.bfloat16)
            acc = jnp.zeros((b, d), dtype=jnp.float32)
            for t in range(4):
                acc = acc + term(l, t, xb)
            for ti in range(2):
                for c in grp[(2, l, ti)]:
                    c.wait_recv()
            for t in range(4, N_DEV):
                convert(l, t)
            for t in range(4, N_DEV):
                acc = acc + term(l, t, xb)
            x_cur = acc
        out_ref[:, :] = x_cur

        for c in sends:
            c.wait_send()

    return pl.pallas_call(
        body,
        out_shape=jax.ShapeDtypeStruct((b, d), jnp.float32),
        in_specs=[pl.BlockSpec(memory_space=pltpu.VMEM)] * 7,
        out_specs=pl.BlockSpec(memory_space=pltpu.VMEM),
        scratch_shapes=[
            pltpu.VMEM((N_LAYERS, N_DEV, d, h), jnp.int8),
            pltpu.VMEM((N_LAYERS, N_DEV, h, d), jnp.int8),
            pltpu.VMEM((N_LAYERS, N_DEV, d // _BLK, h), jnp.bfloat16),
            pltpu.VMEM((N_LAYERS, N_DEV, h // _BLK, d), jnp.bfloat16),
            pltpu.VMEM((N_LAYERS, N_DEV, d, h), jnp.bfloat16),
            pltpu.VMEM((N_LAYERS, N_DEV, h, d), jnp.bfloat16),
            pltpu.SemaphoreType.DMA((N_LAYERS,)),
            pltpu.SemaphoreType.DMA((N_LAYERS,)),
            pltpu.SemaphoreType.DMA((3, N_LAYERS)),
            pltpu.SemaphoreType.DMA((3, N_LAYERS)),
        ],
        compiler_params=pltpu.CompilerParams(collective_id=0),
    )(x, Win0, Wout0, Win1, Wout1, Win2, Wout2)
